# baseline (speedup 1.0000x reference)
import jax
import jax.numpy as jnp
from jax import lax
from jax.experimental import pallas as pl
from jax.experimental.pallas import tpu as pltpu
from jax.experimental.pallas import tpu_sc as plsc

_D = 8
_I = 16
_E = 4
_T = 16384
_J = _E * _I
_NC = 2
_NS = 16
_NW = _NC * _NS
_TPW = _T // _NW
_CAP = 576
_HF = 4


def _rb16(v):
    i = plsc.bitcast(v, jnp.int32)
    lsb = lax.shift_right_logical(i, 16) & 1
    r = (i + 0x7FFF + lsb) & jnp.int32(-65536)
    return plsc.bitcast(r, jnp.float32)


def _rb16f(v):
    i = plsc.bitcast(v, jnp.int32)
    r = (i + 0x8000) & jnp.int32(-65536)
    return plsc.bitcast(r, jnp.float32)


def _tec_body(x_hbm, rw_hbm, ws_hbm, gt_hbm, ut_hbm, dt_hbm,
              gs_hbm, us_hbm, ds_hbm, out_hbm,
              x_v, out_v, rw_v, ws_v, g_v, u_v, d_v, s_v, ids_v, wl_v):
    wid = lax.axis_index("s") * _NC + lax.axis_index("c")
    base = wid * (_TPW * _D)

    pltpu.sync_copy(x_hbm.at[pl.ds(base, _TPW * _D)], x_v)
    pltpu.sync_copy(rw_hbm, rw_v)
    pltpu.sync_copy(ws_hbm, ws_v)
    pltpu.sync_copy(gt_hbm, g_v)
    pltpu.sync_copy(ut_hbm, u_v)
    pltpu.sync_copy(dt_hbm, d_v)

    pltpu.sync_copy(gs_hbm, s_v)
    for k in range(_J * _D // 16):
        sl = pl.ds(k * 16, 16)
        g_v[sl] = _rb16(g_v[sl] * s_v[sl])
    pltpu.sync_copy(us_hbm, s_v)
    for k in range(_J * _D // 16):
        sl = pl.ds(k * 16, 16)
        u_v[sl] = _rb16(u_v[sl] * s_v[sl])
    pltpu.sync_copy(ds_hbm, s_v)
    for k in range(_J * _D // 16):
        sl = pl.ds(k * 16, 16)
        d_v[sl] = _rb16(d_v[sl] * s_v[sl])
    zi = jnp.zeros((16,), jnp.int32)
    for k in range(_E * _CAP // 16):
        ids_v[pl.ds(k * 16, 16)] = zi

    iota = lax.iota(jnp.int32, 16)

    def phase1(t, c):
        toff = t * 16
        rows = toff + iota
        rows8 = rows * _D
        xd = [plsc.load_gather(x_v, [rows8 + dd]) for dd in range(_D)]
        rw = [rw_v[pl.ds(k * 16, 16)] for k in range(_E * _D // 16)]
        l = []
        for e in range(_E):
            rvec = rw[(e * _D) // 16]
            off = (e * _D) % 16
            a = rvec[off] * xd[0]
            for dd in range(1, _D):
                a = a + rvec[off + dd] * xd[dd]
            l.append(a)
        v1 = l[0]
        a1 = jnp.zeros((16,), jnp.int32)
        for e in range(1, _E):
            cnd = l[e] > v1
            v1 = jnp.where(cnd, l[e], v1)
            a1 = jnp.where(cnd, jnp.full((16,), e, jnp.int32), a1)
        v2 = jnp.full((16,), -jnp.inf, jnp.float32)
        a2 = jnp.zeros((16,), jnp.int32)
        for e in range(_E):
            cnd = jnp.logical_and(l[e] > v2, a1 != e)
            v2 = jnp.where(cnd, l[e], v2)
            a2 = jnp.where(cnd, jnp.full((16,), e, jnp.int32), a2)
        ed = jnp.exp(v2 - v1)
        w1 = 0.5 / (1.0 + ed)
        w2 = 0.5 - w1
        wsv = [ws_v[pl.ds(k * 16, 16)] for k in range(_D * _D // 16)]
        for dd in range(_D):
            wvec = wsv[(dd * _D) // 16]
            off = (dd * _D) % 16
            a = wvec[off] * xd[0]
            for d2 in range(1, _D):
                a = a + wvec[off + d2] * xd[d2]
            plsc.store_scatter(out_v, [rows8 + dd], a)
        cs = []
        for e in range(_E):
            m1 = a1 == e
            m2 = a2 == e
            m = jnp.logical_or(m1, m2)
            wv = jnp.where(m1, w1, w2)
            be = e * _CAP + c[e]
            plsc.store_compressed(ids_v.at[pl.ds(be, 16)], rows, mask=m)
            plsc.store_compressed(wl_v.at[pl.ds(be, 16)], wv, mask=m)
            cnt = plsc.all_reduce_population_count(m)
            cs.append(c[e] + cnt[0])
        return tuple(cs)

    z = jnp.int32(0)
    counts = lax.fori_loop(0, _TPW // 16, phase1, (z, z, z, z))

    for e in range(_E):
        n_e = counts[e]

        def chunk(p, _, e=e, n_e=n_e):
            off0 = p * (_HF * 16)
            offm = e * _CAP + off0
            ids = [ids_v[pl.ds(offm + hf * 16, 16)] for hf in range(_HF)]
            wv = [wl_v[pl.ds(offm + hf * 16, 16)] for hf in range(_HF)]
            mk = [iota < (n_e - (off0 + hf * 16)) for hf in range(_HF)]
            idx8 = [ids[hf] * _D for hf in range(_HF)]
            xd = [[plsc.load_gather(x_v, [idx8[hf] + dd]) for dd in range(_D)]
                  for hf in range(_HF)]
            acc = [[None] * _D for _ in range(_HF)]
            for jj in range(_I // 2):
                row = e * _I * _D + jj * 16
                gv = g_v[pl.ds(row, 16)]
                uv = u_v[pl.ds(row, 16)]
                dv = d_v[pl.ds(row, 16)]
                for h2 in range(2):
                    o = h2 * _D
                    g2 = [None] * _HF
                    u2 = [None] * _HF
                    for dd in range(_D):
                        w = gv[o + dd]
                        for hf in range(_HF):
                            a = w * xd[hf][dd]
                            g2[hf] = a if g2[hf] is None else g2[hf] + a
                        w = uv[o + dd]
                        for hf in range(_HF):
                            a = w * xd[hf][dd]
                            u2[hf] = a if u2[hf] is None else u2[hf] + a
                    h2v = [_rb16f((g2[hf] / (1.0 + jnp.exp(-g2[hf])))
                                  * u2[hf]) * wv[hf] for hf in range(_HF)]
                    for dd in range(_D):
                        w = dv[o + dd]
                        for hf in range(_HF):
                            a = w * h2v[hf]
                            acc[hf][dd] = (a if acc[hf][dd] is None
                                           else acc[hf][dd] + a)
            for hf in range(_HF):
                for dd in range(_D):
                    plsc.addupdate_scatter(out_v, [idx8[hf] + dd],
                                           acc[hf][dd], mask=mk[hf])
            return 0

        npairs = lax.div(n_e + (_HF * 16 - 1), jnp.int32(_HF * 16))
        lax.fori_loop(0, npairs, chunk, 0)

    pltpu.sync_copy(out_v, out_hbm.at[pl.ds(base, _TPW * _D)])


def _pre_round_bf16(v):
    i = lax.bitcast_convert_type(v, jnp.int32)
    lsb = lax.shift_right_logical(i, 16) & 1
    r = (i + 0x7FFF + lsb) & jnp.int32(-65536)
    return lax.bitcast_convert_type(r, jnp.float32)


@jax.jit
def kernel(x, router_weight, shared_W, gate_s, up_s, down_s,
           gate_w, up_w, down_w):
    xf = _pre_round_bf16(x.reshape(_T * _D))
    rwf = _pre_round_bf16(router_weight.reshape(_E * _D))
    wsf = _pre_round_bf16(shared_W.reshape(_D * _D))
    gtf = gate_w.astype(jnp.float32).reshape(_J * _D)
    utf = up_w.astype(jnp.float32).reshape(_J * _D)
    dtf = jnp.transpose(down_w, (0, 2, 1)).astype(jnp.float32).reshape(_J * _D)
    gsf = jnp.broadcast_to(gate_s.reshape(_E, _I, 1), (_E, _I, _D)).reshape(_J * _D)
    usf = jnp.broadcast_to(up_s.reshape(_E, _I, 1), (_E, _I, _D)).reshape(_J * _D)
    dsf = jnp.broadcast_to(down_s.reshape(_E, 1, _D), (_E, _I, _D)).reshape(_J * _D)

    mesh = plsc.VectorSubcoreMesh(core_axis_name="c", subcore_axis_name="s",
                                  num_cores=_NC, num_subcores=_NS)
    run = pl.kernel(
        _tec_body,
        out_type=jax.ShapeDtypeStruct((_T * _D,), jnp.float32),
        mesh=mesh,
        compiler_params=pltpu.CompilerParams(needs_layout_passes=False),
        scratch_types=[
            pltpu.VMEM((_TPW * _D,), jnp.float32),
            pltpu.VMEM((_TPW * _D,), jnp.float32),
            pltpu.VMEM((_E * _D,), jnp.float32),
            pltpu.VMEM((_D * _D,), jnp.float32),
            pltpu.VMEM((_J * _D,), jnp.float32),
            pltpu.VMEM((_J * _D,), jnp.float32),
            pltpu.VMEM((_J * _D,), jnp.float32),
            pltpu.VMEM((_J * _D,), jnp.float32),
            pltpu.VMEM((_E * _CAP,), jnp.int32),
            pltpu.VMEM((_E * _CAP,), jnp.float32),
        ],
    )
    out = run(xf, rwf, wsf, gtf, utf, dtf, gsf, usf, dsf)
    return out.reshape(_T, _D)


# speedup vs baseline: 1.5979x; 1.5979x over previous
import jax
import jax.numpy as jnp
from jax import lax
from jax.experimental import pallas as pl
from jax.experimental.pallas import tpu as pltpu
from jax.experimental.pallas import tpu_sc as plsc

_D = 8
_I = 16
_E = 4
_T = 16384
_J = _E * _I
_NC = 2
_NS = 16
_NW = _NC * _NS
_TPW = _T // _NW
_CAP = 560
_HF = 3

_RW_O = 0
_WS_O = _RW_O + _E * _D
_G_O = _WS_O + _D * _D
_U_O = _G_O + _J * _D
_DW_O = _U_O + _J * _D
_GS_O = _DW_O + _J * _D
_US_O = _GS_O + _J * _D
_DS_O = _US_O + _J * _D
_WB = _DS_O + _J * _D


def _rb16(v):
    i = plsc.bitcast(v, jnp.int32)
    lsb = lax.shift_right_logical(i, 16) & 1
    r = (i + 0x7FFF + lsb) & jnp.int32(-65536)
    return plsc.bitcast(r, jnp.float32)


def _rb16f(v):
    i = plsc.bitcast(v, jnp.int32)
    r = (i + 0x8000) & jnp.int32(-65536)
    return plsc.bitcast(r, jnp.float32)


def _tec_body(x_hbm, wb_hbm, out_hbm,
              x_v, out_v, wb_v, ids_v, wl_v):
    wid = lax.axis_index("s") * _NC + lax.axis_index("c")
    base = wid * (_TPW * _D)

    pltpu.sync_copy(x_hbm.at[pl.ds(base, _TPW * _D)], x_v)
    pltpu.sync_copy(wb_hbm, wb_v)

    for k in range(_J * _D // 16):
        sl = pl.ds(_G_O + k * 16, 16)
        ss = pl.ds(_GS_O + k * 16, 16)
        wb_v[sl] = _rb16(wb_v[sl] * wb_v[ss])
    for k in range(_J * _D // 16):
        sl = pl.ds(_U_O + k * 16, 16)
        ss = pl.ds(_US_O + k * 16, 16)
        wb_v[sl] = _rb16(wb_v[sl] * wb_v[ss])
    for k in range(_J * _D // 16):
        sl = pl.ds(_DW_O + k * 16, 16)
        ss = pl.ds(_DS_O + k * 16, 16)
        wb_v[sl] = _rb16(wb_v[sl] * wb_v[ss])
    zi = jnp.zeros((16,), jnp.int32)
    for k in range(_E * _CAP // 16):
        ids_v[pl.ds(k * 16, 16)] = zi

    iota = lax.iota(jnp.int32, 16)

    def phase1(t, c):
        toff = t * 16
        rows = toff + iota
        rows8 = rows * _D
        xd = [plsc.load_gather(x_v, [rows8 + dd]) for dd in range(_D)]
        rw = [wb_v[pl.ds(_RW_O + k * 16, 16)] for k in range(_E * _D // 16)]
        l = []
        for e in range(_E):
            rvec = rw[(e * _D) // 16]
            off = (e * _D) % 16
            a = rvec[off] * xd[0]
            for dd in range(1, _D):
                a = a + rvec[off + dd] * xd[dd]
            l.append(a)
        v1 = l[0]
        a1 = jnp.zeros((16,), jnp.int32)
        for e in range(1, _E):
            cnd = l[e] > v1
            v1 = jnp.where(cnd, l[e], v1)
            a1 = jnp.where(cnd, jnp.full((16,), e, jnp.int32), a1)
        v2 = jnp.full((16,), -jnp.inf, jnp.float32)
        a2 = jnp.zeros((16,), jnp.int32)
        for e in range(_E):
            cnd = jnp.logical_and(l[e] > v2, a1 != e)
            v2 = jnp.where(cnd, l[e], v2)
            a2 = jnp.where(cnd, jnp.full((16,), e, jnp.int32), a2)
        ed = jnp.exp(v2 - v1)
        w1 = 0.5 / (1.0 + ed)
        w2 = 0.5 - w1
        wsv = [wb_v[pl.ds(_WS_O + k * 16, 16)] for k in range(_D * _D // 16)]
        for dd in range(_D):
            wvec = wsv[(dd * _D) // 16]
            off = (dd * _D) % 16
            a = wvec[off] * xd[0]
            for d2 in range(1, _D):
                a = a + wvec[off + d2] * xd[d2]
            plsc.store_scatter(out_v, [rows8 + dd], a)
        cs = []
        for e in range(_E):
            m1 = a1 == e
            m2 = a2 == e
            m = jnp.logical_or(m1, m2)
            wv = jnp.where(m1, w1, w2)
            be = e * _CAP + c[e]
            plsc.store_compressed(ids_v.at[pl.ds(be, 16)], rows, mask=m)
            plsc.store_compressed(wl_v.at[pl.ds(be, 16)], wv, mask=m)
            cnt = plsc.all_reduce_population_count(m)
            cs.append(c[e] + cnt[0])
        return tuple(cs)

    z = jnp.int32(0)
    counts = lax.fori_loop(0, _TPW // 16, phase1, (z, z, z, z))

    for e in range(_E):
        n_e = counts[e]

        def chunk(p, _, e=e, n_e=n_e):
            off0 = p * (_HF * 16)
            offm = e * _CAP + off0
            ids = [ids_v[pl.ds(offm + hf * 16, 16)] for hf in range(_HF)]
            wv = [wl_v[pl.ds(offm + hf * 16, 16)] for hf in range(_HF)]
            mk = [iota < (n_e - (off0 + hf * 16)) for hf in range(_HF)]
            idx8 = [ids[hf] * _D for hf in range(_HF)]
            xd = [[plsc.load_gather(x_v, [idx8[hf] + dd]) for dd in range(_D)]
                  for hf in range(_HF)]
            acc = [[None] * _D for _ in range(_HF)]
            for jj in range(_I // 2):
                row = e * _I * _D + jj * 16
                gv = wb_v[pl.ds(_G_O + row, 16)]
                uv = wb_v[pl.ds(_U_O + row, 16)]
                dv = wb_v[pl.ds(_DW_O + row, 16)]
                for h2 in range(2):
                    o = h2 * _D
                    g2 = [None] * _HF
                    u2 = [None] * _HF
                    for dd in range(_D):
                        w = gv[o + dd]
                        for hf in range(_HF):
                            a = w * xd[hf][dd]
                            g2[hf] = a if g2[hf] is None else g2[hf] + a
                        w = uv[o + dd]
                        for hf in range(_HF):
                            a = w * xd[hf][dd]
                            u2[hf] = a if u2[hf] is None else u2[hf] + a
                    h2v = [_rb16f((g2[hf] / (1.0 + jnp.exp(-g2[hf])))
                                  * u2[hf]) * wv[hf] for hf in range(_HF)]
                    for dd in range(_D):
                        w = dv[o + dd]
                        for hf in range(_HF):
                            a = w * h2v[hf]
                            acc[hf][dd] = (a if acc[hf][dd] is None
                                           else acc[hf][dd] + a)
            for hf in range(_HF):
                for dd in range(_D):
                    plsc.addupdate_scatter(out_v, [idx8[hf] + dd],
                                           acc[hf][dd], mask=mk[hf])
            return 0

        npairs = lax.div(n_e + (_HF * 16 - 1), jnp.int32(_HF * 16))
        lax.fori_loop(0, npairs, chunk, 0)

    pltpu.sync_copy(out_v, out_hbm.at[pl.ds(base, _TPW * _D)])


def _pre_round_bf16(v):
    i = lax.bitcast_convert_type(v, jnp.int32)
    lsb = lax.shift_right_logical(i, 16) & 1
    r = (i + 0x7FFF + lsb) & jnp.int32(-65536)
    return lax.bitcast_convert_type(r, jnp.float32)


@jax.jit
def kernel(x, router_weight, shared_W, gate_s, up_s, down_s,
           gate_w, up_w, down_w):
    xf = _pre_round_bf16(x.reshape(_T * _D))
    rwf = _pre_round_bf16(router_weight.reshape(_E * _D))
    wsf = _pre_round_bf16(shared_W.reshape(_D * _D))
    gtf = gate_w.astype(jnp.float32).reshape(_J * _D)
    utf = up_w.astype(jnp.float32).reshape(_J * _D)
    dtf = jnp.transpose(down_w, (0, 2, 1)).astype(jnp.float32).reshape(_J * _D)
    gsf = jnp.broadcast_to(gate_s.reshape(_E, _I, 1), (_E, _I, _D)).reshape(_J * _D)
    usf = jnp.broadcast_to(up_s.reshape(_E, _I, 1), (_E, _I, _D)).reshape(_J * _D)
    dsf = jnp.broadcast_to(down_s.reshape(_E, 1, _D), (_E, _I, _D)).reshape(_J * _D)
    wbf = jnp.concatenate([rwf, wsf, gtf, utf, dtf, gsf, usf, dsf])

    mesh = plsc.VectorSubcoreMesh(core_axis_name="c", subcore_axis_name="s",
                                  num_cores=_NC, num_subcores=_NS)
    run = pl.kernel(
        _tec_body,
        out_type=jax.ShapeDtypeStruct((_T * _D,), jnp.float32),
        mesh=mesh,
        compiler_params=pltpu.CompilerParams(needs_layout_passes=False),
        scratch_types=[
            pltpu.VMEM((_TPW * _D,), jnp.float32),
            pltpu.VMEM((_TPW * _D,), jnp.float32),
            pltpu.VMEM((_WB,), jnp.float32),
            pltpu.VMEM((_E * _CAP,), jnp.int32),
            pltpu.VMEM((_E * _CAP,), jnp.float32),
        ],
    )
    out = run(xf, wbf)
    return out.reshape(_T, _D)


# speedup vs baseline: 2.1384x; 1.3382x over previous
import jax
import jax.numpy as jnp
from jax import lax
from jax.experimental import pallas as pl
from jax.experimental.pallas import tpu as pltpu
from jax.experimental.pallas import tpu_sc as plsc

_D = 8
_I = 16
_E = 4
_T = 16384
_J = _E * _I
_NC = 2
_NS = 16
_NW = _NC * _NS
_TPW = _T // _NW
_CAP = 560
_HF = 3

_RW_O = 0
_WS_O = _RW_O + _E * _D
_G_O = _WS_O + _D * _D
_U_O = _G_O + _J * _D
_DW_O = _U_O + _J * _D
_GS_O = _DW_O + _J * _D
_US_O = _GS_O + _J * _D
_DS_O = _US_O + _J * _D
_WB = _DS_O + _J * _D


def _rb16(v):
    i = plsc.bitcast(v, jnp.int32)
    lsb = lax.shift_right_logical(i, 16) & 1
    r = (i + 0x7FFF + lsb) & jnp.int32(-65536)
    return plsc.bitcast(r, jnp.float32)


def _rb16f(v):
    i = plsc.bitcast(v, jnp.int32)
    r = (i + 0x8000) & jnp.int32(-65536)
    return plsc.bitcast(r, jnp.float32)


def _tec_body(x_hbm, wb_hbm, out_hbm,
              x_v, out_v, wb_v, ids_v, wl_v):
    wid = lax.axis_index("s") * _NC + lax.axis_index("c")
    base = wid * (_TPW * _D)

    pltpu.sync_copy(x_hbm.at[pl.ds(base, _TPW * _D)], x_v)
    pltpu.sync_copy(wb_hbm, wb_v)

    for k in range(_J * _D // 16):
        sl = pl.ds(_G_O + k * 16, 16)
        ss = pl.ds(_GS_O + k * 16, 16)
        wb_v[sl] = _rb16(wb_v[sl] * wb_v[ss])
    for k in range(_J * _D // 16):
        sl = pl.ds(_U_O + k * 16, 16)
        ss = pl.ds(_US_O + k * 16, 16)
        wb_v[sl] = _rb16(wb_v[sl] * wb_v[ss])
    for k in range(_J * _D // 16):
        sl = pl.ds(_DW_O + k * 16, 16)
        ss = pl.ds(_DS_O + k * 16, 16)
        wb_v[sl] = _rb16(wb_v[sl] * wb_v[ss])
    zi = jnp.zeros((16,), jnp.int32)
    for k in range(_E * _CAP // 16):
        ids_v[pl.ds(k * 16, 16)] = zi

    iota = lax.iota(jnp.int32, 16)

    def phase1(t, c):
        toff = t * 16
        rows = toff + iota
        xd = [x_v[pl.ds(dd * _TPW + toff, 16)] for dd in range(_D)]
        rw = [wb_v[pl.ds(_RW_O + k * 16, 16)] for k in range(_E * _D // 16)]
        l = []
        for e in range(_E):
            rvec = rw[(e * _D) // 16]
            off = (e * _D) % 16
            a = rvec[off] * xd[0]
            for dd in range(1, _D):
                a = a + rvec[off + dd] * xd[dd]
            l.append(a)
        v1 = l[0]
        a1 = jnp.zeros((16,), jnp.int32)
        for e in range(1, _E):
            cnd = l[e] > v1
            v1 = jnp.where(cnd, l[e], v1)
            a1 = jnp.where(cnd, jnp.full((16,), e, jnp.int32), a1)
        v2 = jnp.full((16,), -jnp.inf, jnp.float32)
        a2 = jnp.zeros((16,), jnp.int32)
        for e in range(_E):
            cnd = jnp.logical_and(l[e] > v2, a1 != e)
            v2 = jnp.where(cnd, l[e], v2)
            a2 = jnp.where(cnd, jnp.full((16,), e, jnp.int32), a2)
        ed = jnp.exp(v2 - v1)
        w1 = 0.5 / (1.0 + ed)
        w2 = 0.5 - w1
        wsv = [wb_v[pl.ds(_WS_O + k * 16, 16)] for k in range(_D * _D // 16)]
        for dd in range(_D):
            wvec = wsv[(dd * _D) // 16]
            off = (dd * _D) % 16
            a = wvec[off] * xd[0]
            for d2 in range(1, _D):
                a = a + wvec[off + d2] * xd[d2]
            out_v[pl.ds(dd * _TPW + toff, 16)] = a
        cs = []
        for e in range(_E):
            m1 = a1 == e
            m2 = a2 == e
            m = jnp.logical_or(m1, m2)
            wv = jnp.where(m1, w1, w2)
            be = e * _CAP + c[e]
            plsc.store_compressed(ids_v.at[pl.ds(be, 16)], rows, mask=m)
            plsc.store_compressed(wl_v.at[pl.ds(be, 16)], wv, mask=m)
            cnt = plsc.all_reduce_population_count(m)
            cs.append(c[e] + cnt[0])
        return tuple(cs)

    z = jnp.int32(0)
    counts = lax.fori_loop(0, _TPW // 16, phase1, (z, z, z, z))

    for e in range(_E):
        n_e = counts[e]

        def chunk(p, _, e=e, n_e=n_e):
            off0 = p * (_HF * 16)
            offm = e * _CAP + off0
            ids = [ids_v[pl.ds(offm + hf * 16, 16)] for hf in range(_HF)]
            wv = [wl_v[pl.ds(offm + hf * 16, 16)] for hf in range(_HF)]
            mk = [iota < (n_e - (off0 + hf * 16)) for hf in range(_HF)]
            idxd = [[ids[hf] + dd * _TPW for dd in range(_D)]
                    for hf in range(_HF)]
            xd = [[plsc.load_gather(x_v, [idxd[hf][dd]]) for dd in range(_D)]
                  for hf in range(_HF)]
            acc = [[None] * _D for _ in range(_HF)]
            for jj in range(_I // 2):
                row = e * _I * _D + jj * 16
                gv = wb_v[pl.ds(_G_O + row, 16)]
                uv = wb_v[pl.ds(_U_O + row, 16)]
                dv = wb_v[pl.ds(_DW_O + row, 16)]
                for h2 in range(2):
                    o = h2 * _D
                    g2 = [None] * _HF
                    u2 = [None] * _HF
                    for dd in range(_D):
                        w = gv[o + dd]
                        for hf in range(_HF):
                            a = w * xd[hf][dd]
                            g2[hf] = a if g2[hf] is None else g2[hf] + a
                        w = uv[o + dd]
                        for hf in range(_HF):
                            a = w * xd[hf][dd]
                            u2[hf] = a if u2[hf] is None else u2[hf] + a
                    h2v = [_rb16f((g2[hf] / (1.0 + jnp.exp(-g2[hf])))
                                  * u2[hf]) * wv[hf] for hf in range(_HF)]
                    for dd in range(_D):
                        w = dv[o + dd]
                        for hf in range(_HF):
                            a = w * h2v[hf]
                            acc[hf][dd] = (a if acc[hf][dd] is None
                                           else acc[hf][dd] + a)
            for hf in range(_HF):
                for dd in range(_D):
                    plsc.addupdate_scatter(out_v, [idxd[hf][dd]],
                                           acc[hf][dd], mask=mk[hf])
            return 0

        npairs = lax.div(n_e + (_HF * 16 - 1), jnp.int32(_HF * 16))
        lax.fori_loop(0, npairs, chunk, 0)

    pltpu.sync_copy(out_v, out_hbm.at[pl.ds(base, _TPW * _D)])


def _pre_round_bf16(v):
    i = lax.bitcast_convert_type(v, jnp.int32)
    lsb = lax.shift_right_logical(i, 16) & 1
    r = (i + 0x7FFF + lsb) & jnp.int32(-65536)
    return lax.bitcast_convert_type(r, jnp.float32)


@jax.jit
def kernel(x, router_weight, shared_W, gate_s, up_s, down_s,
           gate_w, up_w, down_w):
    xf = _pre_round_bf16(
        jnp.transpose(x.reshape(_NW, _TPW, _D), (0, 2, 1)).reshape(_T * _D))
    rwf = _pre_round_bf16(router_weight.reshape(_E * _D))
    wsf = _pre_round_bf16(shared_W.reshape(_D * _D))
    gtf = gate_w.astype(jnp.float32).reshape(_J * _D)
    utf = up_w.astype(jnp.float32).reshape(_J * _D)
    dtf = jnp.transpose(down_w, (0, 2, 1)).astype(jnp.float32).reshape(_J * _D)
    gsf = jnp.broadcast_to(gate_s.reshape(_E, _I, 1), (_E, _I, _D)).reshape(_J * _D)
    usf = jnp.broadcast_to(up_s.reshape(_E, _I, 1), (_E, _I, _D)).reshape(_J * _D)
    dsf = jnp.broadcast_to(down_s.reshape(_E, 1, _D), (_E, _I, _D)).reshape(_J * _D)
    wbf = jnp.concatenate([rwf, wsf, gtf, utf, dtf, gsf, usf, dsf])

    mesh = plsc.VectorSubcoreMesh(core_axis_name="c", subcore_axis_name="s",
                                  num_cores=_NC, num_subcores=_NS)
    run = pl.kernel(
        _tec_body,
        out_type=jax.ShapeDtypeStruct((_T * _D,), jnp.float32),
        mesh=mesh,
        compiler_params=pltpu.CompilerParams(needs_layout_passes=False),
        scratch_types=[
            pltpu.VMEM((_TPW * _D,), jnp.float32),
            pltpu.VMEM((_TPW * _D,), jnp.float32),
            pltpu.VMEM((_WB,), jnp.float32),
            pltpu.VMEM((_E * _CAP,), jnp.int32),
            pltpu.VMEM((_E * _CAP,), jnp.float32),
        ],
    )
    out = run(xf, wbf)
    return jnp.transpose(out.reshape(_NW, _D, _TPW), (0, 2, 1)).reshape(_T, _D)
